# baseline (speedup 1.0000x reference)
import functools
import jax
import jax.numpy as jnp
from jax import lax
from jax.experimental import pallas as pl
from jax.experimental.pallas import tpu as pltpu, tpu_sc as plsc

NCH = 384
ROWS = 25088
NW = 32
RPW = ROWS // NW
R = 112
NCHUNK = RPW // R
NG = NCH // 16


def _make_sc_kernel():
    mesh = plsc.VectorSubcoreMesh(core_axis_name="c", subcore_axis_name="s")

    @functools.partial(
        pl.kernel,
        mesh=mesh,
        out_type=jax.ShapeDtypeStruct((ROWS, NCH), jnp.float32),
        scratch_types=[
            pltpu.VMEM((R, NCH), jnp.float32),
            pltpu.VMEM((R, NCH), jnp.float32),
            pltpu.SemaphoreType.DMA,
            pltpu.SemaphoreType.DMA,
            pltpu.SemaphoreType.DMA,
            pltpu.SemaphoreType.DMA,
        ],
    )
    def k(x_hbm, o_hbm, buf0, buf1, gsem0, gsem1, ssem0, ssem1):
        wid = lax.axis_index("s") * 2 + lax.axis_index("c")
        base = wid * RPW

        bufs = (buf0, buf1)
        gsems = (gsem0, gsem1)
        ssems = (ssem0, ssem1)

        def load(j):
            lo = pl.multiple_of(base + j * R, 8)
            return pltpu.make_async_copy(
                x_hbm.at[pl.ds(lo, R)], bufs[j % 2], gsems[j % 2]
            )

        def store(j):
            lo = pl.multiple_of(base + j * R, 8)
            return pltpu.make_async_copy(
                bufs[j % 2], o_hbm.at[pl.ds(lo, R)], ssems[j % 2]
            )

        def reverse_lanes(buf):
            def body(i2, _):
                for u in range(2):
                    i = i2 * 2 + u
                    for g in range(NG // 2):
                        lo = pl.ds(16 * g, 16)
                        hi = pl.ds(NCH - 16 * (g + 1), 16)
                        t0 = buf[i, lo]
                        t1 = buf[i, hi]
                        buf[i, lo] = lax.rev(t1, (0,))
                        buf[i, hi] = lax.rev(t0, (0,))
                return _

            lax.fori_loop(0, R // 2, body, None)

        load(0).start()
        for j in range(NCHUNK):
            if j + 1 < NCHUNK:
                if j >= 1:
                    store(j - 1).wait()
                load(j + 1).start()
            load(j).wait()
            reverse_lanes(bufs[j % 2])
            store(j).start()
        store(NCHUNK - 2).wait()
        store(NCHUNK - 1).wait()

    return k


_sc_kernel = _make_sc_kernel()


def kernel(input):
    b, c, h, w = input.shape
    xt = jnp.transpose(input, (0, 2, 3, 1)).reshape(b * h * w, c)
    out = _sc_kernel(xt)
    return jnp.transpose(out.reshape(b, h, w, c), (0, 3, 1, 2))


# speedup vs baseline: 1.4502x; 1.4502x over previous
import functools
import jax
import jax.numpy as jnp
from jax import lax
from jax.experimental import pallas as pl
from jax.experimental.pallas import tpu as pltpu, tpu_sc as plsc

NCH = 384
ROWS = 25088
NW = 32
RPW = ROWS // NW
R = 56
NCHUNK = RPW // R
NG = NCH // 16


def _make_sc_kernel():
    mesh = plsc.VectorSubcoreMesh(core_axis_name="c", subcore_axis_name="s")

    @functools.partial(
        pl.kernel,
        mesh=mesh,
        out_type=jax.ShapeDtypeStruct((ROWS, NCH), jnp.float32),
        scratch_types=[
            pltpu.VMEM((R, NCH), jnp.float32),
            pltpu.VMEM((R, NCH), jnp.float32),
            pltpu.SemaphoreType.DMA,
            pltpu.SemaphoreType.DMA,
            pltpu.SemaphoreType.DMA,
            pltpu.SemaphoreType.DMA,
        ],
    )
    def k(x_hbm, o_hbm, buf0, buf1, gsem0, gsem1, ssem0, ssem1):
        wid = lax.axis_index("s") * 2 + lax.axis_index("c")
        base = wid * RPW

        bufs = (buf0, buf1)
        gsems = (gsem0, gsem1)
        ssems = (ssem0, ssem1)

        def load(j):
            lo = pl.multiple_of(base + j * R, 8)
            return pltpu.make_async_copy(
                x_hbm.at[pl.ds(lo, R)], bufs[j % 2], gsems[j % 2]
            )

        def store(j):
            lo = pl.multiple_of(base + j * R, 8)
            return pltpu.make_async_copy(
                bufs[j % 2], o_hbm.at[pl.ds(lo, R)], ssems[j % 2]
            )

        def reverse_lanes(buf):
            def body(i, _):
                for g in range(NG // 2):
                    lo = pl.ds(16 * g, 16)
                    hi = pl.ds(NCH - 16 * (g + 1), 16)
                    t0 = buf[i, lo]
                    t1 = buf[i, hi]
                    buf[i, lo] = lax.rev(t1, (0,))
                    buf[i, hi] = lax.rev(t0, (0,))
                return _

            lax.fori_loop(0, R, body, None)

        load(0).start()
        for j in range(NCHUNK):
            if j + 1 < NCHUNK:
                if j >= 1:
                    store(j - 1).wait()
                load(j + 1).start()
            load(j).wait()
            reverse_lanes(bufs[j % 2])
            store(j).start()
        store(NCHUNK - 2).wait()
        store(NCHUNK - 1).wait()

    return k


_sc_kernel = _make_sc_kernel()


def kernel(input):
    b, c, h, w = input.shape
    xt = jnp.transpose(input, (0, 2, 3, 1)).reshape(b * h * w, c)
    out = _sc_kernel(xt)
    return jnp.transpose(out.reshape(b, h, w, c), (0, 3, 1, 2))


# speedup vs baseline: 1.4818x; 1.0217x over previous
import functools
import jax
import jax.numpy as jnp
from jax import lax
from jax.experimental import pallas as pl
from jax.experimental.pallas import tpu as pltpu, tpu_sc as plsc

NCH = 384
ROWS = 25088
NW = 32
RPW = ROWS // NW
R = 112
NCHUNK = RPW // R
NG = NCH // 16


def _make_sc_kernel():
    mesh = plsc.VectorSubcoreMesh(core_axis_name="c", subcore_axis_name="s")

    @functools.partial(
        pl.kernel,
        mesh=mesh,
        out_type=jax.ShapeDtypeStruct((ROWS, NCH), jnp.float32),
        scratch_types=[
            pltpu.VMEM((R, NCH), jnp.float32),
            pltpu.VMEM((R, NCH), jnp.float32),
            pltpu.SemaphoreType.DMA,
            pltpu.SemaphoreType.DMA,
            pltpu.SemaphoreType.DMA,
            pltpu.SemaphoreType.DMA,
        ],
    )
    def k(x_hbm, o_hbm, buf0, buf1, gsem0, gsem1, ssem0, ssem1):
        wid = lax.axis_index("s") * 2 + lax.axis_index("c")
        base = wid * RPW

        bufs = (buf0, buf1)
        gsems = (gsem0, gsem1)
        ssems = (ssem0, ssem1)

        def load(j):
            lo = pl.multiple_of(base + j * R, 8)
            return pltpu.make_async_copy(
                x_hbm.at[pl.ds(lo, R)], bufs[j % 2], gsems[j % 2]
            )

        def store(j):
            lo = pl.multiple_of(base + j * R, 8)
            return pltpu.make_async_copy(
                bufs[j % 2], o_hbm.at[pl.ds(lo, R)], ssems[j % 2]
            )

        def reverse_lanes(buf):
            def body(i, _):
                for g in range(NG // 2):
                    lo = pl.ds(16 * g, 16)
                    hi = pl.ds(NCH - 16 * (g + 1), 16)
                    t0 = buf[i, lo]
                    t1 = buf[i, hi]
                    buf[i, lo] = lax.rev(t1, (0,))
                    buf[i, hi] = lax.rev(t0, (0,))
                return _

            lax.fori_loop(0, R, body, None)

        load(0).start()
        for j in range(NCHUNK):
            if j + 1 < NCHUNK:
                if j >= 1:
                    store(j - 1).wait()
                load(j + 1).start()
            load(j).wait()
            reverse_lanes(bufs[j % 2])
            store(j).start()
        store(NCHUNK - 2).wait()
        store(NCHUNK - 1).wait()

    return k


_sc_kernel = _make_sc_kernel()


def kernel(input):
    b, c, h, w = input.shape
    xt = jnp.transpose(input, (0, 2, 3, 1)).reshape(b * h * w, c)
    out = _sc_kernel(xt)
    return jnp.transpose(out.reshape(b, h, w, c), (0, 3, 1, 2))


# speedup vs baseline: 1.5773x; 1.0645x over previous
import functools
import jax
import jax.numpy as jnp
from jax import lax
from jax.experimental import pallas as pl
from jax.experimental.pallas import tpu as pltpu, tpu_sc as plsc

NCH = 384
ROWS = 25088
NW = 32
RPW = ROWS // NW
R = 56
NCHUNK = RPW // R
NG = NCH // 16


def _make_sc_kernel():
    mesh = plsc.VectorSubcoreMesh(core_axis_name="c", subcore_axis_name="s")

    @functools.partial(
        pl.kernel,
        mesh=mesh,
        out_type=jax.ShapeDtypeStruct((ROWS, NCH), jnp.float32),
        scratch_types=[
            pltpu.VMEM((R, NCH), jnp.float32),
            pltpu.VMEM((R, NCH), jnp.float32),
            pltpu.VMEM((R, NCH), jnp.float32),
            pltpu.VMEM((R, NCH), jnp.float32),
            pltpu.SemaphoreType.DMA,
            pltpu.SemaphoreType.DMA,
            pltpu.SemaphoreType.DMA,
            pltpu.SemaphoreType.DMA,
        ],
    )
    def k(x_hbm, o_hbm, in0, in1, out0, out1, gsem0, gsem1, ssem0, ssem1):
        wid = lax.axis_index("s") * 2 + lax.axis_index("c")
        base = wid * RPW

        ins = (in0, in1)
        outs = (out0, out1)
        gsems = (gsem0, gsem1)
        ssems = (ssem0, ssem1)

        def load(j):
            lo = pl.multiple_of(base + j * R, 8)
            return pltpu.make_async_copy(
                x_hbm.at[pl.ds(lo, R)], ins[j % 2], gsems[j % 2]
            )

        def store(j):
            lo = pl.multiple_of(base + j * R, 8)
            return pltpu.make_async_copy(
                outs[j % 2], o_hbm.at[pl.ds(lo, R)], ssems[j % 2]
            )

        def reverse_lanes(src, dst):
            @functools.partial(plsc.parallel_loop, 0, R, unroll=2)
            def _loop(i):
                for g in range(NG):
                    t = src[i, pl.ds(NCH - 16 * (g + 1), 16)]
                    dst[i, pl.ds(16 * g, 16)] = lax.rev(t, (0,))

        load(0).start()
        for j in range(NCHUNK):
            s = j % 2
            if j + 1 < NCHUNK:
                load(j + 1).start()
            load(j).wait()
            if j >= 2:
                store(j - 2).wait()
            reverse_lanes(ins[s], outs[s])
            store(j).start()
        store(NCHUNK - 2).wait()
        store(NCHUNK - 1).wait()

    return k


_sc_kernel = _make_sc_kernel()


def kernel(input):
    b, c, h, w = input.shape
    xt = jnp.transpose(input, (0, 2, 3, 1)).reshape(b * h * w, c)
    out = _sc_kernel(xt)
    return jnp.transpose(out.reshape(b, h, w, c), (0, 3, 1, 2))
